# baseline (speedup 1.0000x reference)
import functools

import jax
import jax.numpy as jnp
from jax import lax
from jax.experimental import pallas as pl
from jax.experimental.pallas import tpu as pltpu
from jax.experimental.pallas import tpu_sc as plsc

VOCAB = 100000
EMB_D = 16
BATCH = 1024

_NUM_CORES = 2
_NUM_SUBCORES = 16
_NW = _NUM_CORES * _NUM_SUBCORES
_BPW = BATCH // _NW

N_BLK = 2048
M_BLK = 4096
_PACK_ROWS = N_BLK // 8
_NROWS = pl.cdiv(VOCAB, N_BLK) * _PACK_ROWS


def _repack_body(t_ref, out_ref):
    tt = t_ref[...].T
    out_ref[...] = jnp.concatenate(
        [tt[k * _PACK_ROWS:(k + 1) * _PACK_ROWS, :] for k in range(8)],
        axis=1)


def _repack(table_t):
    return pl.pallas_call(
        _repack_body,
        grid=(pl.cdiv(VOCAB, N_BLK),),
        in_specs=[pl.BlockSpec((EMB_D, N_BLK), lambda i: (0, i))],
        out_specs=pl.BlockSpec((_PACK_ROWS, 128), lambda i: (i, 0)),
        out_shape=jax.ShapeDtypeStruct((_NROWS, 128), jnp.float32),
    )(table_t)


def _sc_gather(packed16, idx):
    mesh = plsc.VectorSubcoreMesh(core_axis_name="c", subcore_axis_name="s")

    @functools.partial(
        pl.kernel,
        mesh=mesh,
        compiler_params=pltpu.CompilerParams(use_tc_tiling_on_sc=False),
        out_type=jax.ShapeDtypeStruct((BATCH, EMB_D), jnp.float32),
        scratch_types=[
            pltpu.VMEM((_BPW,), jnp.int32),
            pltpu.VMEM((_BPW,), jnp.int32),
            pltpu.VMEM((_BPW, EMB_D), jnp.float32),
            pltpu.SemaphoreType.DMA,
        ],
    )
    def k(packed_hbm, idx_hbm, out_hbm, ids_v, rows_i, emb_v, sem):
        wid = lax.axis_index("s") * _NUM_CORES + lax.axis_index("c")
        base = wid * _BPW
        pltpu.sync_copy(idx_hbm.at[pl.ds(base, _BPW)], ids_v)
        for c in range(_BPW // 16):
            v = ids_v[pl.ds(c * 16, 16)]
            rows_i[pl.ds(c * 16, 16)] = (
                ((v >> 11) << 11) + ((v & 255) << 3) + ((v >> 8) & 7))
        pltpu.async_copy(packed_hbm.at[rows_i], emb_v, sem).wait()
        pltpu.sync_copy(emb_v, out_hbm.at[pl.ds(base, _BPW)])

    return k(packed16, idx)


def _mm_t_body(wt_ref, emb_ref, out_ref):
    out_ref[...] = lax.dot_general(
        wt_ref[...], emb_ref[...],
        (((0,), (1,)), ((), ())),
        preferred_element_type=jnp.float32,
    )


def kernel(center_ids, emb_table, W, b):
    ids = center_ids.astype(jnp.int32)
    packed = _repack(emb_table.T)
    emb = _sc_gather(packed.reshape(_NROWS * 8, EMB_D), ids)
    emb_aug = jnp.concatenate(
        [emb, jnp.ones((BATCH, 1), jnp.float32)], axis=1)
    wt_aug = jnp.concatenate([W.T, b[None, :]], axis=0)
    out_t = pl.pallas_call(
        _mm_t_body,
        grid=(pl.cdiv(VOCAB, M_BLK),),
        in_specs=[
            pl.BlockSpec((EMB_D + 1, M_BLK), lambda i: (0, i)),
            pl.BlockSpec((BATCH, EMB_D + 1), lambda i: (0, 0)),
        ],
        out_specs=pl.BlockSpec((M_BLK, BATCH), lambda i: (i, 0)),
        out_shape=jax.ShapeDtypeStruct((VOCAB, BATCH), jnp.float32),
    )(wt_aug, emb_aug)
    return out_t.T


# speedup vs baseline: 1.1930x; 1.1930x over previous
import functools

import jax
import jax.numpy as jnp
from jax import lax
from jax.experimental import pallas as pl
from jax.experimental.pallas import tpu as pltpu
from jax.experimental.pallas import tpu_sc as plsc

VOCAB = 100000
EMB_D = 16
BATCH = 1024

_NUM_CORES = 2
_NUM_SUBCORES = 16
_NW = _NUM_CORES * _NUM_SUBCORES
_BPW = BATCH // _NW

N_BLK = 2048
M_BLK = 2048
_PACK_ROWS = N_BLK // 8
_NROWS = pl.cdiv(VOCAB, N_BLK) * _PACK_ROWS


_RB = 8 * N_BLK
_RSTEPS = pl.cdiv(VOCAB, _RB)


def _repack_body(*refs):
    big = jnp.concatenate([r[...] for r in refs[:8]], axis=0)
    refs[8][...] = big.T


def _repack(table_t):
    table_t = jnp.pad(table_t, ((0, 0), (0, _RSTEPS * _RB - VOCAB)))
    in_specs = [
        pl.BlockSpec((EMB_D, N_BLK), (lambda i, g=g: (0, 8 * i + g)))
        for g in range(8)
    ]
    return pl.pallas_call(
        _repack_body,
        grid=(_RSTEPS,),
        in_specs=in_specs,
        out_specs=pl.BlockSpec((N_BLK, 128), lambda i: (i, 0)),
        out_shape=jax.ShapeDtypeStruct((_RSTEPS * N_BLK, 128), jnp.float32),
    )(*([table_t] * 8))


def _sc_gather(packed16, idx):
    mesh = plsc.VectorSubcoreMesh(core_axis_name="c", subcore_axis_name="s")

    @functools.partial(
        pl.kernel,
        mesh=mesh,
        compiler_params=pltpu.CompilerParams(use_tc_tiling_on_sc=False),
        out_type=jax.ShapeDtypeStruct((BATCH, EMB_D), jnp.float32),
        scratch_types=[
            pltpu.VMEM((_BPW,), jnp.int32),
            pltpu.VMEM((_BPW,), jnp.int32),
            pltpu.VMEM((_BPW, EMB_D), jnp.float32),
            pltpu.SemaphoreType.DMA,
        ],
    )
    def k(packed_hbm, idx_hbm, out_hbm, ids_v, rows_i, emb_v, sem):
        wid = lax.axis_index("s") * _NUM_CORES + lax.axis_index("c")
        base = wid * _BPW
        pltpu.sync_copy(idx_hbm.at[pl.ds(base, _BPW)], ids_v)
        for c in range(_BPW // 16):
            v = ids_v[pl.ds(c * 16, 16)]
            rows_i[pl.ds(c * 16, 16)] = (
                ((v >> 14) << 14) + ((v & 2047) << 3) + ((v >> 11) & 7))
        pltpu.async_copy(packed_hbm.at[rows_i], emb_v, sem).wait()
        pltpu.sync_copy(emb_v, out_hbm.at[pl.ds(base, _BPW)])

    return k(packed16, idx)


def _mm_t_body(wt_ref, b_ref, emb_ref, out_ref):
    wt = jnp.concatenate([wt_ref[...], b_ref[...]], axis=0)
    out_ref[...] = lax.dot_general(
        wt, emb_ref[...],
        (((0,), (1,)), ((), ())),
        preferred_element_type=jnp.float32,
    )


def kernel(center_ids, emb_table, W, b):
    ids = center_ids.astype(jnp.int32)
    packed = _repack(emb_table.T)
    emb = _sc_gather(packed.reshape(_RSTEPS * N_BLK * 8, EMB_D), ids)
    emb_aug = jnp.concatenate(
        [emb, jnp.ones((BATCH, 1), jnp.float32)], axis=1)
    out_t = pl.pallas_call(
        _mm_t_body,
        grid=(pl.cdiv(VOCAB, M_BLK),),
        in_specs=[
            pl.BlockSpec((EMB_D, M_BLK), lambda i: (0, i)),
            pl.BlockSpec((1, M_BLK), lambda i: (0, i)),
            pl.BlockSpec((BATCH, EMB_D + 1), lambda i: (0, 0)),
        ],
        out_specs=pl.BlockSpec((M_BLK, BATCH), lambda i: (i, 0)),
        out_shape=jax.ShapeDtypeStruct((VOCAB, BATCH), jnp.float32),
    )(W.T, b.reshape(1, VOCAB), emb_aug)
    return out_t.T
